# Initial kernel scaffold; baseline (speedup 1.0000x reference)
import jax
import jax.numpy as jnp
from jax.experimental import pallas as pl


def kernel(x2d, projected_pix, fov_mask):
    raise NotImplementedError("write your pallas kernel here")


# speedup vs baseline: 6.1085x; 6.1085x over previous
import functools

import jax
import jax.numpy as jnp
from jax import lax
from jax.experimental import pallas as pl
from jax.experimental.pallas import tpu as pltpu, tpu_sc as plsc

_NC = 2
_NS = 16
_NW = _NC * _NS
_L = 16

_C = 128
_H, _W = 96, 320
_HW = _H * _W
_TW = _HW + _L
_N = 128 * 128 * 16
_R = _C // _NW
_PER_S = _N // _NS
_CHP = 512
_CH = 512


def _flosp_body(src, pix, fov, out, idx_out,
                table_v, idx_v, out_v, pix_v, fov_v, idxo_v):
    sc = lax.axis_index("c")
    s = lax.axis_index("s")
    wid = s * _NC + sc
    lane = lax.iota(jnp.int32, _L)

    def p1_chunk(g, carry):
        base = s * _PER_S + g * _CHP
        pltpu.sync_copy(pix.at[pl.ds(2 * base, 2 * _CHP)], pix_v)
        pltpu.sync_copy(fov.at[pl.ds(base, _CHP)], fov_v)

        def p1_vec(j, c2):
            px = plsc.load_gather(pix_v, [j * 32 + 2 * lane])
            py = plsc.load_gather(pix_v, [j * 32 + 2 * lane + 1])
            fv = fov_v[pl.ds(j * _L, _L)]
            ic = jnp.clip(py, 0, _H - 1) * _W + jnp.clip(px, 0, _W - 1)
            ic = jnp.where(fv != 0, ic, _HW)
            idxo_v[pl.ds(j * _L, _L)] = ic
            return c2

        lax.fori_loop(0, _CHP // _L, p1_vec, 0)
        pltpu.sync_copy(idxo_v, idx_out.at[sc, pl.ds(base, _CHP)])
        return carry

    lax.fori_loop(0, _PER_S // _CHP, p1_chunk, 0)
    plsc.subcore_barrier()

    for r in range(_R):
        pltpu.sync_copy(src.at[_R * wid + r], table_v.at[r, pl.ds(0, _HW)])
        table_v[r, pl.ds(_HW, _L)] = jnp.zeros((_L,), jnp.float32)

    row_ids = [jnp.full((_L,), r, jnp.int32) for r in range(_R)]

    def p2_chunk(g, carry):
        pltpu.sync_copy(idx_out.at[sc, pl.ds(g * _CH, _CH)], idx_v)

        def p2_vec(j, c2):
            iv = idx_v[pl.ds(j * _L, _L)]
            for r in range(_R):
                out_v[r, pl.ds(j * _L, _L)] = plsc.load_gather(
                    table_v, [row_ids[r], iv])
            return c2

        lax.fori_loop(0, _CH // _L, p2_vec, 0)
        for r in range(_R):
            pltpu.sync_copy(out_v.at[r],
                            out.at[_R * wid + r, pl.ds(g * _CH, _CH)])
        return carry

    lax.fori_loop(0, _N // _CH, p2_chunk, 0)


@functools.partial(jax.jit, static_argnums=())
def _flosp_gather(src, pix, fov):
    mesh = plsc.VectorSubcoreMesh(core_axis_name="c", subcore_axis_name="s")
    f = pl.kernel(
        _flosp_body,
        out_type=(
            jax.ShapeDtypeStruct((_C, _N), jnp.float32),
            jax.ShapeDtypeStruct((_NC, _N), jnp.int32),
        ),
        mesh=mesh,
        compiler_params=pltpu.CompilerParams(needs_layout_passes=False),
        scratch_types=(
            pltpu.VMEM((_R, _TW), jnp.float32),
            pltpu.VMEM((_CH,), jnp.int32),
            pltpu.VMEM((_R, _CH), jnp.float32),
            pltpu.VMEM((2 * _CHP,), jnp.int32),
            pltpu.VMEM((_CHP,), jnp.int32),
            pltpu.VMEM((_CHP,), jnp.int32),
        ),
    )
    return f(src, pix, fov)


def kernel(x2d, projected_pix, fov_mask):
    bs, c, h, w = x2d.shape
    src = x2d.reshape(c, h * w)
    pix = projected_pix.reshape(-1)
    fov = fov_mask.reshape(-1).astype(jnp.int32)
    out, _ = _flosp_gather(src, pix, fov)
    return out.reshape(bs, c, 128, 128, 16)


# speedup vs baseline: 11.5093x; 1.8842x over previous
import functools

import jax
import jax.numpy as jnp
from jax import lax
from jax.experimental import pallas as pl
from jax.experimental.pallas import tpu as pltpu, tpu_sc as plsc

_NC = 2
_NS = 16
_NW = _NC * _NS
_L = 16

_C = 128
_H, _W = 96, 320
_HW = _H * _W
_TW = _HW + _L
_N = 128 * 128 * 16
_R = _C // _NW
_PER_S = _N // _NS
_CHP = 256
_CH = 512
_NCH = _N // _CH


def _flosp_body(src, pix, fov, out, idx_out,
                table_v, idx_v, out_v, pix_v, fov_v, idxo_v,
                sem_in, sem_out):
    sc = lax.axis_index("c")
    s = lax.axis_index("s")
    wid = s * _NC + sc
    lane = lax.iota(jnp.int32, _L)

    def p1_chunk(g, carry):
        base = s * _PER_S + g * _CHP
        pltpu.sync_copy(pix.at[pl.ds(2 * base, 2 * _CHP)], pix_v)
        pltpu.sync_copy(fov.at[pl.ds(base, _CHP)], fov_v)

        @plsc.parallel_loop(0, _CHP // _L, unroll=4)
        def p1_vec(j):
            px = plsc.load_gather(pix_v, [j * 32 + 2 * lane])
            py = plsc.load_gather(pix_v, [j * 32 + 2 * lane + 1])
            fv = fov_v[pl.ds(j * _L, _L)]
            ic = jnp.clip(py, 0, _H - 1) * _W + jnp.clip(px, 0, _W - 1)
            ic = jnp.where(fv != 0, ic, _HW)
            idxo_v[pl.ds(j * _L, _L)] = ic

        pltpu.sync_copy(idxo_v, idx_out.at[sc, pl.ds(base, _CHP)])
        return carry

    lax.fori_loop(0, _PER_S // _CHP, p1_chunk, 0)
    plsc.subcore_barrier()

    for r in range(_R):
        pltpu.sync_copy(src.at[_R * wid + r], table_v.at[r, pl.ds(0, _HW)])
        table_v[r, pl.ds(_HW, _L)] = jnp.zeros((_L,), jnp.float32)

    row_ids = [jnp.full((_L,), r, jnp.int32) for r in range(_R)]

    def idx_cp(g, par):
        return pltpu.make_async_copy(
            idx_out.at[sc, pl.ds(g * _CH, _CH)], idx_v.at[par], sem_in.at[par])

    def out_cp(g, par, r):
        return pltpu.make_async_copy(
            out_v.at[par, r], out.at[_R * wid + r, pl.ds(g * _CH, _CH)],
            sem_out.at[par])

    idx_cp(0, 0).start()

    def p2_pair(gp, carry):
        g0 = gp * 2
        for par in range(2):
            g = g0 + par

            @pl.when(g + 1 < _NCH)
            def _():
                idx_cp(g + 1, 1 - par).start()

            idx_cp(g, par).wait()

            @pl.when(g >= 2)
            def _():
                for r in range(_R):
                    out_cp(g - 2, par, r).wait()

            @plsc.parallel_loop(0, _CH // _L, unroll=4)
            def p2_vec(j):
                iv = idx_v[par, pl.ds(j * _L, _L)]
                for r in range(_R):
                    out_v[par, r, pl.ds(j * _L, _L)] = plsc.load_gather(
                        table_v, [row_ids[r], iv])

            for r in range(_R):
                out_cp(g, par, r).start()
        return carry

    lax.fori_loop(0, _NCH // 2, p2_pair, 0)
    for par in range(2):
        for r in range(_R):
            out_cp(_NCH - 2 + par, par, r).wait()


@functools.partial(jax.jit, static_argnums=())
def _flosp_gather(src, pix, fov):
    mesh = plsc.VectorSubcoreMesh(core_axis_name="c", subcore_axis_name="s")
    f = pl.kernel(
        _flosp_body,
        out_type=(
            jax.ShapeDtypeStruct((_C, _N), jnp.float32),
            jax.ShapeDtypeStruct((_NC, _N), jnp.int32),
        ),
        mesh=mesh,
        compiler_params=pltpu.CompilerParams(needs_layout_passes=False),
        scratch_types=(
            pltpu.VMEM((_R, _TW), jnp.float32),
            pltpu.VMEM((2, _CH), jnp.int32),
            pltpu.VMEM((2, _R, _CH), jnp.float32),
            pltpu.VMEM((2 * _CHP,), jnp.int32),
            pltpu.VMEM((_CHP,), jnp.int32),
            pltpu.VMEM((_CHP,), jnp.int32),
            pltpu.SemaphoreType.DMA((2,)),
            pltpu.SemaphoreType.DMA((2,)),
        ),
    )
    return f(src, pix, fov)


def kernel(x2d, projected_pix, fov_mask):
    bs, c, h, w = x2d.shape
    src = x2d.reshape(c, h * w)
    pix = projected_pix.reshape(-1)
    fov = fov_mask.reshape(-1).astype(jnp.int32)
    out, _ = _flosp_gather(src, pix, fov)
    return out.reshape(bs, c, 128, 128, 16)


# speedup vs baseline: 11.6254x; 1.0101x over previous
import functools

import jax
import jax.numpy as jnp
from jax import lax
from jax.experimental import pallas as pl
from jax.experimental.pallas import tpu as pltpu, tpu_sc as plsc

_NC = 2
_NS = 16
_NW = _NC * _NS
_L = 16

_C = 128
_H, _W = 96, 320
_HW = _H * _W
_TW = _HW + _L
_N = 128 * 128 * 16
_R = _C // _NW
_PER_S = _N // _NS
_CHP = 256
_CH = 512
_NCH = _N // _CH


def _flosp_body(src, pix, fov, out, idx_out,
                table_v, idx_v, out_v, pix_v, fov_v, idxo_v,
                sem_in, sem_out):
    sc = lax.axis_index("c")
    s = lax.axis_index("s")
    wid = s * _NC + sc
    lane = lax.iota(jnp.int32, _L)

    def p1_chunk(g, carry):
        base = s * _PER_S + g * _CHP
        pltpu.sync_copy(pix.at[pl.ds(2 * base, 2 * _CHP)], pix_v)
        pltpu.sync_copy(fov.at[pl.ds(base, _CHP)], fov_v)

        @plsc.parallel_loop(0, _CHP // _L, unroll=4)
        def p1_vec(j):
            px = plsc.load_gather(pix_v, [j * 32 + 2 * lane])
            py = plsc.load_gather(pix_v, [j * 32 + 2 * lane + 1])
            fv = fov_v[pl.ds(j * _L, _L)]
            ic = jnp.clip(py, 0, _H - 1) * _W + jnp.clip(px, 0, _W - 1)
            ic = jnp.where(fv != 0, ic, _HW)
            idxo_v[pl.ds(j * _L, _L)] = ic

        pltpu.sync_copy(idxo_v, idx_out.at[sc, pl.ds(base, _CHP)])
        return carry

    lax.fori_loop(0, _PER_S // _CHP, p1_chunk, 0)
    plsc.subcore_barrier()

    for r in range(_R):
        pltpu.sync_copy(src.at[_R * wid + r], table_v.at[r, pl.ds(0, _HW)])
        table_v[r, pl.ds(_HW, _L)] = jnp.zeros((_L,), jnp.float32)

    row_ids = [jnp.full((_L,), r, jnp.int32) for r in range(_R)]

    def idx_cp(g, par):
        return pltpu.make_async_copy(
            idx_out.at[sc, pl.ds(g * _CH, _CH)], idx_v.at[par], sem_in.at[par])

    def out_cp(g, par):
        return pltpu.make_async_copy(
            out_v.at[par],
            out.at[pl.ds(_R * wid, _R), pl.ds(g * _CH, _CH)],
            sem_out.at[par])

    idx_cp(0, 0).start()

    def p2_pair(gp, carry):
        g0 = gp * 2
        for par in range(2):
            g = g0 + par

            @pl.when(g + 1 < _NCH)
            def _():
                idx_cp(g + 1, 1 - par).start()

            idx_cp(g, par).wait()

            @pl.when(g >= 2)
            def _():
                out_cp(g - 2, par).wait()

            @plsc.parallel_loop(0, _CH // _L, unroll=4)
            def p2_vec(j):
                iv = idx_v[par, pl.ds(j * _L, _L)]
                for r in range(_R):
                    out_v[par, r, pl.ds(j * _L, _L)] = plsc.load_gather(
                        table_v, [row_ids[r], iv])

            out_cp(g, par).start()
        return carry

    lax.fori_loop(0, _NCH // 2, p2_pair, 0)
    for par in range(2):
        out_cp(_NCH - 2 + par, par).wait()


@functools.partial(jax.jit, static_argnums=())
def _flosp_gather(src, pix, fov):
    mesh = plsc.VectorSubcoreMesh(core_axis_name="c", subcore_axis_name="s")
    f = pl.kernel(
        _flosp_body,
        out_type=(
            jax.ShapeDtypeStruct((_C, _N), jnp.float32),
            jax.ShapeDtypeStruct((_NC, _N), jnp.int32),
        ),
        mesh=mesh,
        compiler_params=pltpu.CompilerParams(needs_layout_passes=False),
        scratch_types=(
            pltpu.VMEM((_R, _TW), jnp.float32),
            pltpu.VMEM((2, _CH), jnp.int32),
            pltpu.VMEM((2, _R, _CH), jnp.float32),
            pltpu.VMEM((2 * _CHP,), jnp.int32),
            pltpu.VMEM((_CHP,), jnp.int32),
            pltpu.VMEM((_CHP,), jnp.int32),
            pltpu.SemaphoreType.DMA((2,)),
            pltpu.SemaphoreType.DMA((2,)),
        ),
    )
    return f(src, pix, fov)


def kernel(x2d, projected_pix, fov_mask):
    bs, c, h, w = x2d.shape
    src = x2d.reshape(c, h * w)
    pix = projected_pix.reshape(-1)
    fov = fov_mask.reshape(-1).astype(jnp.int32)
    out, _ = _flosp_gather(src, pix, fov)
    return out.reshape(bs, c, 128, 128, 16)
